# baseline (speedup 1.0000x reference)
import functools

import jax
import jax.numpy as jnp
from jax import lax
from jax.experimental import pallas as pl
from jax.experimental.pallas import tpu as pltpu
from jax.experimental.pallas import tpu_sc as plsc

_NC = 2
_NS = 16
_NW = _NC * _NS
_L = 16

_D = 2048
_H = 10
_GROUP = 16
_SUB = 4


def _tc_kernel(x_ref, w1_ref, b1_ref, w2_ref, b2_ref, o_ref):
    x = x_ref[...]
    scores = jnp.dot(x, w1_ref[...], preferred_element_type=jnp.float32)
    scores = scores + b1_ref[...]
    R, H = scores.shape
    col = lax.broadcasted_iota(jnp.int32, (R, H), 1)
    m1 = jnp.max(scores, axis=1, keepdims=True)
    i1 = jnp.min(jnp.where(scores == m1, col, H), axis=1, keepdims=True)
    mask1 = col == i1
    rest = jnp.where(mask1, -jnp.inf, scores)
    m2 = jnp.max(rest, axis=1, keepdims=True)
    i2 = jnp.min(jnp.where(rest == m2, col, H), axis=1, keepdims=True)
    sel = jnp.where(mask1 | (col == i2), scores, 0.0)
    o_ref[...] = jnp.dot(sel, w2_ref[...], preferred_element_type=jnp.float32) + b2_ref[...]


def _tc_run(events2d, W1, b1, W2, b2, tc_rows, block_rows):
    n_rows, d = events2d.shape
    h = W1.shape[1]
    grid = (tc_rows // block_rows,)
    return pl.pallas_call(
        _tc_kernel,
        grid=grid,
        in_specs=[
            pl.BlockSpec((block_rows, d), lambda i: (i, 0)),
            pl.BlockSpec((d, h), lambda i: (0, 0)),
            pl.BlockSpec((1, h), lambda i: (0, 0)),
            pl.BlockSpec((h, 1), lambda i: (0, 0)),
            pl.BlockSpec((1, 1), lambda i: (0, 0)),
        ],
        out_specs=pl.BlockSpec((block_rows, 1), lambda i: (i, 0)),
        out_shape=jax.ShapeDtypeStruct((tc_rows, 1), jnp.float32),
    )(events2d, W1, b1.reshape(1, h), W2, b2.reshape(1, 1))


def _round_bf16(v):
    u = plsc.bitcast(v, jnp.uint32)
    r = u + jnp.uint32(0x7FFF) + ((u >> jnp.uint32(16)) & jnp.uint32(1))
    return plsc.bitcast(r & jnp.uint32(0xFFFF0000), jnp.float32)

def _sc_make(sc_base, sc_rows):
    rpw = sc_rows // _NW
    n_groups = rpw // _GROUP

    mesh = plsc.VectorSubcoreMesh(core_axis_name="c", subcore_axis_name="s",
                                  num_cores=_NC, num_subcores=_NS)

    @functools.partial(
        pl.kernel,
        out_type=jax.ShapeDtypeStruct((sc_rows, _L), jnp.float32),
        mesh=mesh,
        compiler_params=pltpu.CompilerParams(needs_layout_passes=False),
        scratch_types=[
            pltpu.VMEM((_H, _D), jnp.float32),
            pltpu.VMEM((16, 16), jnp.float32),
            pltpu.VMEM((_GROUP, _D), jnp.float32),
            pltpu.VMEM((rpw, _L), jnp.float32),
        ],
    )
    def sc_kernel(ev_hbm, w1t_hbm, cons_hbm, out_hbm, w1t_v, sm_v, x_v, out_v):
        wid = lax.axis_index("s") * _NC + lax.axis_index("c")
        pltpu.sync_copy(w1t_hbm, w1t_v)
        pltpu.sync_copy(cons_hbm, sm_v.at[pl.ds(0, 3)])
        iota = lax.iota(jnp.int32, _L)
        zero = jnp.zeros((_L,), jnp.float32)
        b1p = sm_v[0, :]
        w2p = sm_v[1, :]
        b2l0 = sm_v[2, :]

        def group_body(g, carry):
            r0 = sc_base + wid * rpw + g * _GROUP
            pltpu.sync_copy(ev_hbm.at[pl.ds(r0, _GROUP), :], x_v)
            for sb in range(_GROUP // _SUB):
                def j_body(j, accs):
                    accs = list(accs)
                    for r in range(_SUB):
                        xv = _round_bf16(x_v[sb * _SUB + r, pl.ds(j * _L, _L)])
                        for h in range(_H):
                            wv = w1t_v[h, pl.ds(j * _L, _L)]
                            accs[r * _H + h] = accs[r * _H + h] + xv * wv
                    return tuple(accs)

                accs = lax.fori_loop(
                    0, _D // _L, j_body,
                    tuple(zero for _ in range(_SUB * _H)))
                for r in range(_SUB):
                    part = zero
                    for h in range(_H):
                        s_h = jnp.sum(accs[r * _H + h])
                        part = jnp.where(iota == h,
                                         jnp.broadcast_to(s_h, (_L,)), part)
                    score = part + b1p
                    m1 = jnp.max(score)
                    i1 = plsc.all_reduce_ffs(score == m1)
                    mask1 = iota == i1
                    rest = jnp.where(mask1, -3.0e38, score)
                    m2 = jnp.max(rest)
                    i2 = plsc.all_reduce_ffs(rest == m2)
                    sel = jnp.where(mask1 | (iota == i2), score, 0.0)
                    sel = _round_bf16(sel)
                    s = jnp.sum(sel * w2p + b2l0)
                    out_v[g * _GROUP + sb * _SUB + r, :] = jnp.where(
                        iota == 0, jnp.broadcast_to(s, (_L,)), zero)
            return carry

        lax.fori_loop(0, n_groups, group_body, 0)
        pltpu.sync_copy(out_v, out_hbm.at[pl.ds(wid * rpw, rpw), :])

    return sc_kernel


_SC_ROWS = 4096
_TC_BLOCK = 2048


@jax.jit
def _run(events2d, W1, b1, W2, b2):
    n_rows, d = events2d.shape
    h = W1.shape[1]
    tc_rows = n_rows - _SC_ROWS
    out_tc = _tc_run(events2d, W1, b1, W2, b2, tc_rows, _TC_BLOCK)
    w1t = lax.reduce_precision(W1.T, 8, 7)
    w2r = lax.reduce_precision(W2[:, 0], 8, 7)
    pad = jnp.zeros((_L - h,), jnp.float32)
    cons = jnp.stack([
        jnp.concatenate([b1, pad - 3.0e38]),
        jnp.concatenate([w2r, pad]),
        jnp.concatenate([b2, jnp.zeros((_L - 1,), jnp.float32)]),
    ])
    out_sc = _sc_make(tc_rows, _SC_ROWS)(events2d, w1t, cons)
    return jnp.concatenate([out_tc, out_sc[:, :1]], axis=0)


def kernel(events, W1, b1, W2, b2):
    B, S, D = events.shape
    out = _run(events.reshape(B * S, D), W1, b1, W2, b2)
    return out.reshape(B, S, 1)


# speedup vs baseline: 1.0927x; 1.0927x over previous
import functools

import jax
import jax.numpy as jnp
from jax import lax
from jax.experimental import pallas as pl
from jax.experimental.pallas import tpu as pltpu
from jax.experimental.pallas import tpu_sc as plsc

_NC = 2
_NS = 16
_NW = _NC * _NS
_L = 16

_D = 2048
_H = 10
_GROUP = 16
_SUB = 4
_UNROLL = 2


def _tc_kernel(x_ref, w1_ref, b1_ref, w2_ref, b2_ref, o_ref):
    x = x_ref[...]
    scores = jnp.dot(x, w1_ref[...], preferred_element_type=jnp.float32)
    scores = scores + b1_ref[...]
    R, H = scores.shape
    col = lax.broadcasted_iota(jnp.int32, (R, H), 1)
    m1 = jnp.max(scores, axis=1, keepdims=True)
    i1 = jnp.min(jnp.where(scores == m1, col, H), axis=1, keepdims=True)
    mask1 = col == i1
    rest = jnp.where(mask1, -jnp.inf, scores)
    m2 = jnp.max(rest, axis=1, keepdims=True)
    i2 = jnp.min(jnp.where(rest == m2, col, H), axis=1, keepdims=True)
    sel = jnp.where(mask1 | (col == i2), scores, 0.0)
    o_ref[...] = jnp.dot(sel, w2_ref[...], preferred_element_type=jnp.float32) + b2_ref[...]


def _tc_run(events2d, W1, b1, W2, b2, tc_rows, block_rows):
    n_rows, d = events2d.shape
    h = W1.shape[1]
    grid = (tc_rows // block_rows,)
    return pl.pallas_call(
        _tc_kernel,
        grid=grid,
        in_specs=[
            pl.BlockSpec((block_rows, d), lambda i: (i, 0)),
            pl.BlockSpec((d, h), lambda i: (0, 0)),
            pl.BlockSpec((1, h), lambda i: (0, 0)),
            pl.BlockSpec((h, 1), lambda i: (0, 0)),
            pl.BlockSpec((1, 1), lambda i: (0, 0)),
        ],
        out_specs=pl.BlockSpec((block_rows, 1), lambda i: (i, 0)),
        out_shape=jax.ShapeDtypeStruct((tc_rows, 1), jnp.float32),
    )(events2d, W1, b1.reshape(1, h), W2, b2.reshape(1, 1))


def _round_bf16(v):
    u = plsc.bitcast(v, jnp.uint32)
    r = u + jnp.uint32(0x7FFF) + ((u >> jnp.uint32(16)) & jnp.uint32(1))
    return plsc.bitcast(r & jnp.uint32(0xFFFF0000), jnp.float32)

def _sc_make(sc_base, sc_rows):
    rpw = sc_rows // _NW
    n_groups = rpw // _GROUP

    mesh = plsc.VectorSubcoreMesh(core_axis_name="c", subcore_axis_name="s",
                                  num_cores=_NC, num_subcores=_NS)

    @functools.partial(
        pl.kernel,
        out_type=jax.ShapeDtypeStruct((sc_rows, _L), jnp.float32),
        mesh=mesh,
        compiler_params=pltpu.CompilerParams(needs_layout_passes=False),
        scratch_types=[
            pltpu.VMEM((_H, _D), jnp.float32),
            pltpu.VMEM((16, 16), jnp.float32),
            pltpu.VMEM((_GROUP, _D), jnp.float32),
            pltpu.VMEM((rpw, _L), jnp.float32),
        ],
    )
    def sc_kernel(ev_hbm, w1t_hbm, cons_hbm, out_hbm, w1t_v, sm_v, x_v, out_v):
        wid = lax.axis_index("s") * _NC + lax.axis_index("c")
        pltpu.sync_copy(w1t_hbm, w1t_v)
        pltpu.sync_copy(cons_hbm, sm_v.at[pl.ds(0, 3)])
        iota = lax.iota(jnp.int32, _L)
        zero = jnp.zeros((_L,), jnp.float32)
        b1p = sm_v[0, :]
        w2p = sm_v[1, :]
        b2l0 = sm_v[2, :]

        def group_body(g, carry):
            r0 = sc_base + wid * rpw + g * _GROUP
            pltpu.sync_copy(ev_hbm.at[pl.ds(r0, _GROUP), :], x_v)
            for sb in range(_GROUP // _SUB):
                def j_body(j, accs):
                    accs = list(accs)
                    for u in range(_UNROLL):
                        jj = j * _UNROLL + u
                        for r in range(_SUB):
                            xv = _round_bf16(
                                x_v[sb * _SUB + r, pl.ds(jj * _L, _L)])
                            for h in range(_H):
                                wv = w1t_v[h, pl.ds(jj * _L, _L)]
                                accs[r * _H + h] = accs[r * _H + h] + xv * wv
                    return tuple(accs)

                accs = lax.fori_loop(
                    0, _D // (_L * _UNROLL), j_body,
                    tuple(zero for _ in range(_SUB * _H)))
                for r in range(_SUB):
                    part = zero
                    for h in range(_H):
                        s_h = jnp.sum(accs[r * _H + h])
                        part = jnp.where(iota == h,
                                         jnp.broadcast_to(s_h, (_L,)), part)
                    score = part + b1p
                    m1 = jnp.max(score)
                    i1 = plsc.all_reduce_ffs(score == m1)
                    mask1 = iota == i1
                    rest = jnp.where(mask1, -3.0e38, score)
                    m2 = jnp.max(rest)
                    i2 = plsc.all_reduce_ffs(rest == m2)
                    sel = jnp.where(mask1 | (iota == i2), score, 0.0)
                    sel = _round_bf16(sel)
                    s = jnp.sum(sel * w2p + b2l0)
                    out_v[g * _GROUP + sb * _SUB + r, :] = jnp.where(
                        iota == 0, jnp.broadcast_to(s, (_L,)), zero)
            return carry

        lax.fori_loop(0, n_groups, group_body, 0)
        pltpu.sync_copy(out_v, out_hbm.at[pl.ds(wid * rpw, rpw), :])

    return sc_kernel


_SC_ROWS = 2048
_TC_BLOCK = 2048


@jax.jit
def _run(events2d, W1, b1, W2, b2):
    n_rows, d = events2d.shape
    h = W1.shape[1]
    tc_rows = n_rows - _SC_ROWS
    out_tc = _tc_run(events2d, W1, b1, W2, b2, tc_rows, _TC_BLOCK)
    w1t = lax.reduce_precision(W1.T, 8, 7)
    w2r = lax.reduce_precision(W2[:, 0], 8, 7)
    pad = jnp.zeros((_L - h,), jnp.float32)
    cons = jnp.stack([
        jnp.concatenate([b1, pad - 3.0e38]),
        jnp.concatenate([w2r, pad]),
        jnp.concatenate([b2, jnp.zeros((_L - 1,), jnp.float32)]),
    ])
    out_sc = _sc_make(tc_rows, _SC_ROWS)(events2d, w1t, cons)
    return jnp.concatenate([out_tc, out_sc[:, :1]], axis=0)


def kernel(events, W1, b1, W2, b2):
    B, S, D = events.shape
    out = _run(events.reshape(B * S, D), W1, b1, W2, b2)
    return out.reshape(B, S, 1)


# speedup vs baseline: 1.0932x; 1.0004x over previous
import functools

import jax
import jax.numpy as jnp
from jax import lax
from jax.experimental import pallas as pl
from jax.experimental.pallas import tpu as pltpu
from jax.experimental.pallas import tpu_sc as plsc

_NC = 2
_NS = 16
_NW = _NC * _NS
_L = 16

_D = 2048
_H = 10
_GROUP = 16
_SUB = 4
_UNROLL = 2


def _tc_kernel(x_ref, w1_ref, b1_ref, w2_ref, b2_ref, o_ref):
    x = x_ref[...]
    scores = jnp.dot(x, w1_ref[...], preferred_element_type=jnp.float32)
    scores = scores + b1_ref[...]
    R, H = scores.shape
    col = lax.broadcasted_iota(jnp.int32, (R, H), 1)
    m1 = jnp.max(scores, axis=1, keepdims=True)
    i1 = jnp.min(jnp.where(scores == m1, col, H), axis=1, keepdims=True)
    mask1 = col == i1
    rest = jnp.where(mask1, -jnp.inf, scores)
    m2 = jnp.max(rest, axis=1, keepdims=True)
    i2 = jnp.min(jnp.where(rest == m2, col, H), axis=1, keepdims=True)
    sel = jnp.where(mask1 | (col == i2), scores, 0.0)
    o_ref[...] = jnp.dot(sel, w2_ref[...], preferred_element_type=jnp.float32) + b2_ref[...]


def _tc_run(events2d, W1, b1, W2, b2, tc_rows, block_rows):
    n_rows, d = events2d.shape
    h = W1.shape[1]
    grid = (tc_rows // block_rows,)
    return pl.pallas_call(
        _tc_kernel,
        grid=grid,
        in_specs=[
            pl.BlockSpec((block_rows, d), lambda i: (i, 0)),
            pl.BlockSpec((d, h), lambda i: (0, 0)),
            pl.BlockSpec((1, h), lambda i: (0, 0)),
            pl.BlockSpec((h, 1), lambda i: (0, 0)),
            pl.BlockSpec((1, 1), lambda i: (0, 0)),
        ],
        out_specs=pl.BlockSpec((block_rows, 1), lambda i: (i, 0)),
        out_shape=jax.ShapeDtypeStruct((tc_rows, 1), jnp.float32),
    )(events2d, W1, b1.reshape(1, h), W2, b2.reshape(1, 1))


def _round_bf16(v):
    u = plsc.bitcast(v, jnp.uint32)
    r = u + jnp.uint32(0x7FFF) + ((u >> jnp.uint32(16)) & jnp.uint32(1))
    return plsc.bitcast(r & jnp.uint32(0xFFFF0000), jnp.float32)

def _sc_make(sc_base, sc_rows):
    rpw = sc_rows // _NW
    n_groups = rpw // _GROUP

    mesh = plsc.VectorSubcoreMesh(core_axis_name="c", subcore_axis_name="s",
                                  num_cores=_NC, num_subcores=_NS)

    @functools.partial(
        pl.kernel,
        out_type=jax.ShapeDtypeStruct((sc_rows, _L), jnp.float32),
        mesh=mesh,
        compiler_params=pltpu.CompilerParams(needs_layout_passes=False),
        scratch_types=[
            pltpu.VMEM((_H, _D), jnp.float32),
            pltpu.VMEM((16, 16), jnp.float32),
            pltpu.VMEM((_GROUP, _D), jnp.float32),
            pltpu.VMEM((rpw, _L), jnp.float32),
        ],
    )
    def sc_kernel(ev_hbm, w1t_hbm, cons_hbm, out_hbm, w1t_v, sm_v, x_v, out_v):
        wid = lax.axis_index("s") * _NC + lax.axis_index("c")
        pltpu.sync_copy(w1t_hbm, w1t_v)
        pltpu.sync_copy(cons_hbm, sm_v.at[pl.ds(0, 3)])
        iota = lax.iota(jnp.int32, _L)
        zero = jnp.zeros((_L,), jnp.float32)
        b1p = sm_v[0, :]
        w2p = sm_v[1, :]
        b2l0 = sm_v[2, :]

        def group_body(g, carry):
            r0 = sc_base + wid * rpw + g * _GROUP
            pltpu.sync_copy(ev_hbm.at[pl.ds(r0, _GROUP), :], x_v)
            for sb in range(_GROUP // _SUB):
                def j_body(j, accs):
                    accs = list(accs)
                    for u in range(_UNROLL):
                        jj = j * _UNROLL + u
                        for r in range(_SUB):
                            xv = _round_bf16(
                                x_v[sb * _SUB + r, pl.ds(jj * _L, _L)])
                            for h in range(_H):
                                wv = w1t_v[h, pl.ds(jj * _L, _L)]
                                accs[r * _H + h] = accs[r * _H + h] + xv * wv
                    return tuple(accs)

                accs = lax.fori_loop(
                    0, _D // (_L * _UNROLL), j_body,
                    tuple(zero for _ in range(_SUB * _H)))
                for r in range(_SUB):
                    part = zero
                    for h in range(_H):
                        s_h = jnp.sum(accs[r * _H + h])
                        part = jnp.where(iota == h,
                                         jnp.broadcast_to(s_h, (_L,)), part)
                    score = part + b1p
                    m1 = jnp.max(score)
                    i1 = plsc.all_reduce_ffs(score == m1)
                    mask1 = iota == i1
                    rest = jnp.where(mask1, -3.0e38, score)
                    m2 = jnp.max(rest)
                    i2 = plsc.all_reduce_ffs(rest == m2)
                    sel = jnp.where(mask1 | (iota == i2), score, 0.0)
                    sel = _round_bf16(sel)
                    s = jnp.sum(sel * w2p + b2l0)
                    out_v[g * _GROUP + sb * _SUB + r, :] = jnp.where(
                        iota == 0, jnp.broadcast_to(s, (_L,)), zero)
            return carry

        lax.fori_loop(0, n_groups, group_body, 0)
        pltpu.sync_copy(out_v, out_hbm.at[pl.ds(wid * rpw, rpw), :])

    return sc_kernel


_SC_ROWS = 2048
_TC_BLOCK = 2048


@jax.jit
def _run(events2d, W1, b1, W2, b2):
    n_rows, d = events2d.shape
    h = W1.shape[1]
    tc_rows = n_rows - _SC_ROWS
    w1t = lax.reduce_precision(W1.T, 8, 7)
    w2r = lax.reduce_precision(W2[:, 0], 8, 7)
    pad = jnp.zeros((_L - h,), jnp.float32)
    cons = jnp.stack([
        jnp.concatenate([b1, pad - 3.0e38]),
        jnp.concatenate([w2r, pad]),
        jnp.concatenate([b2, jnp.zeros((_L - 1,), jnp.float32)]),
    ])
    out_sc = _sc_make(tc_rows, _SC_ROWS)(events2d, w1t, cons)
    out_tc = _tc_run(events2d, W1, b1, W2, b2, tc_rows, _TC_BLOCK)
    return jnp.concatenate([out_tc, out_sc[:, :1]], axis=0)


def kernel(events, W1, b1, W2, b2):
    B, S, D = events.shape
    out = _run(events.reshape(B * S, D), W1, b1, W2, b2)
    return out.reshape(B, S, 1)


# speedup vs baseline: 1.4184x; 1.2975x over previous
import functools

import jax
import jax.numpy as jnp
from jax import lax
from jax.experimental import pallas as pl


def _fused_kernel(x_ref, w1_ref, b1_ref, w2_ref, b2_ref, o_ref):
    x = x_ref[...]
    scores = jnp.dot(x, w1_ref[...], preferred_element_type=jnp.float32)
    scores = scores + b1_ref[...]
    R, H = scores.shape
    col = lax.broadcasted_iota(jnp.int32, (R, H), 1)
    m1 = jnp.max(scores, axis=1, keepdims=True)
    i1 = jnp.min(jnp.where(scores == m1, col, H), axis=1, keepdims=True)
    mask1 = col == i1
    rest = jnp.where(mask1, -jnp.inf, scores)
    m2 = jnp.max(rest, axis=1, keepdims=True)
    i2 = jnp.min(jnp.where(rest == m2, col, H), axis=1, keepdims=True)
    sel = jnp.where(mask1 | (col == i2), scores, 0.0)
    o_ref[...] = jnp.dot(sel, w2_ref[...], preferred_element_type=jnp.float32) + b2_ref[...]


@functools.partial(jax.jit, static_argnames=("block_rows",))
def _run(events2d, W1, b1, W2, b2, block_rows):
    n_rows, d = events2d.shape
    h = W1.shape[1]
    grid = (n_rows // block_rows,)
    return pl.pallas_call(
        _fused_kernel,
        grid=grid,
        in_specs=[
            pl.BlockSpec((block_rows, d), lambda i: (i, 0)),
            pl.BlockSpec((d, h), lambda i: (0, 0)),
            pl.BlockSpec((1, h), lambda i: (0, 0)),
            pl.BlockSpec((h, 1), lambda i: (0, 0)),
            pl.BlockSpec((1, 1), lambda i: (0, 0)),
        ],
        out_specs=pl.BlockSpec((block_rows, 1), lambda i: (i, 0)),
        out_shape=jax.ShapeDtypeStruct((n_rows, 1), jnp.float32),
    )(events2d, W1, b1.reshape(1, h), W2, b2.reshape(1, 1))


def kernel(events, W1, b1, W2, b2):
    B, S, D = events.shape
    n_rows = B * S
    block_rows = 2048 if n_rows % 2048 == 0 else 8
    out = _run(events.reshape(n_rows, D), W1, b1, W2, b2, block_rows)
    return out.reshape(B, S, 1)
